# baseline (speedup 1.0000x reference)
import functools

import jax
import jax.numpy as jnp
from jax import lax
from jax.experimental import pallas as pl
from jax.experimental.pallas import tpu as pltpu
from jax.experimental.pallas import tpu_sc as plsc

_GROUPS = ((0, 1), (1, 4), (4, 10))
_R, _A, _C = 4, 10, 8
_D = _R * _A * _C
_DP = 384
_L = 16
_NW = 32
_BLK = 64
_TPB = _BLK // 8 * 24
_NBUF = 3


def _expand_params(p):
    parts = [jnp.broadcast_to(p[g][:, None, :], (_R, e - s, _C))
             for g, (s, e) in enumerate(_GROUPS)]
    return jnp.concatenate(parts, axis=1).reshape(_D)


def _make_sc_call(E, N):
    nblk = E // _BLK
    nk_hi = -(-nblk // _NW)
    nk_lo = nblk // _NW
    nk_pad = -(-nk_hi // _NBUF) * _NBUF
    smax = nk_hi * _BLK
    mesh = plsc.VectorSubcoreMesh(core_axis_name="c", subcore_axis_name="s")
    njc = _D // _L

    @functools.partial(
        pl.kernel,
        mesh=mesh,
        compiler_params=pltpu.CompilerParams(use_tc_tiling_on_sc=False),
        out_type=jax.ShapeDtypeStruct((E // 8 * 24, 128), jnp.float32),
        scratch_types=(
            [pltpu.VMEM((2, 96), jnp.int32) for _ in range(_NBUF)]
            + [pltpu.VMEM((_TPB, 128), jnp.float32) for _ in range(_NBUF)]
            + [pltpu.VMEM((smax,), jnp.float32),
               pltpu.VMEM((smax,), jnp.float32)]
            + [pltpu.VMEM((_D,), jnp.float32), pltpu.VMEM((_D,), jnp.float32)]
            + [pltpu.SemaphoreType.DMA for _ in range(2 * _NBUF)]
        ),
    )
    def sc_kernel(table, gidx, el, cf, inv, pre, out, *refs):
        pidx = refs[0:_NBUF]
        scr = refs[_NBUF:2 * _NBUF]
        sel, scf = refs[2 * _NBUF], refs[2 * _NBUF + 1]
        inv_v, pre_v = refs[2 * _NBUF + 2], refs[2 * _NBUF + 3]
        gsem = refs[2 * _NBUF + 4:2 * _NBUF + 4 + _NBUF]
        osem = refs[2 * _NBUF + 4 + _NBUF:]

        wid = lax.axis_index("s") * 2 + lax.axis_index("c")
        s_w = (wid * nblk) // _NW
        s_n = ((wid + 1) * nblk) // _NW
        nk = s_n - s_w
        ebase_w = s_w * _BLK

        pltpu.sync_copy(inv, inv_v)
        pltpu.sync_copy(pre, pre_v)
        inv_vecs = [inv_v[pl.ds(j * _L, _L)] for j in range(njc)]
        pre_vecs = [pre_v[pl.ds(j * _L, _L)] for j in range(njc)]

        @pl.when(nk == nk_hi)
        def _():
            pltpu.sync_copy(el.at[pl.ds(ebase_w, nk_hi * _BLK)],
                            sel.at[pl.ds(0, nk_hi * _BLK)])
            pltpu.sync_copy(cf.at[pl.ds(ebase_w, nk_hi * _BLK)],
                            scf.at[pl.ds(0, nk_hi * _BLK)])

        @pl.when(nk < nk_hi)
        def _():
            pltpu.sync_copy(el.at[pl.ds(ebase_w, nk_lo * _BLK)],
                            sel.at[pl.ds(0, nk_lo * _BLK)])
            pltpu.sync_copy(cf.at[pl.ds(ebase_w, nk_lo * _BLK)],
                            scf.at[pl.ds(0, nk_lo * _BLK)])

        def load_block(k, b):
            ibase = (s_w + k) * _TPB
            pltpu.sync_copy(gidx.at[pl.ds(ibase, 96)], pidx[b].at[0])
            pltpu.sync_copy(gidx.at[pl.ds(ibase + 96, 96)], pidx[b].at[1])
            pltpu.async_copy(table.at[pidx[b].at[0]],
                             scr[b].at[pl.ds(0, 96)], gsem[b])
            pltpu.async_copy(table.at[pidx[b].at[1]],
                             scr[b].at[pl.ds(96, 96)], gsem[b])

        def wait_gather(b):
            pltpu.make_async_copy(
                table.at[pidx[b].at[0]], scr[b].at[pl.ds(0, 96)],
                gsem[b]).wait()
            pltpu.make_async_copy(
                table.at[pidx[b].at[1]], scr[b].at[pl.ds(96, 96)],
                gsem[b]).wait()

        def compute_block(k, b):
            def group_body(g, c2):
                gbase = g * _L
                el_vec = sel[pl.ds(k * _BLK + gbase, _L)]
                cf_vec = scf[pl.ds(k * _BLK + gbase, _L)]
                for e_l in range(_L):
                    ei = jnp.full((_L,), e_l, jnp.int32)
                    nel = -el_vec.at[ei].get(mode="promise_in_bounds")
                    cf_b = cf_vec.at[ei].get(mode="promise_in_bounds")
                    e = gbase + e_l
                    rbase = (e // 8) * 24 + (e % 8)
                    for j in range(njc):
                        f = j * _L
                        row = rbase + (f // 128) * 8
                        sl = pl.ds(f % 128, _L)
                        s = jnp.exp(nel * inv_vecs[j]) * (pre_vecs[j] * cf_b)
                        scr[b][row, sl] = scr[b][row, sl] * s
                return c2

            lax.fori_loop(0, _BLK // _L, group_body, 0)
            pltpu.async_copy(
                scr[b], out.at[pl.ds((s_w + k) * _TPB, _TPB)], osem[b])

        load_block(0, 0)

        def outer(k3, carry):
            for joff in range(_NBUF):
                k = k3 * _NBUF + joff
                b = joff
                bn = (joff + 1) % _NBUF

                @pl.when(k + 1 < nk)
                def _(k=k, b=b, bn=bn):
                    @pl.when(k >= 2)
                    def _():
                        pltpu.make_async_copy(
                            scr[bn], out.at[pl.ds(0, _TPB)], osem[bn]).wait()

                    load_block(k + 1, bn)

                @pl.when(k < nk)
                def _(k=k, b=b):
                    wait_gather(b)
                    compute_block(k, b)
            return carry

        lax.fori_loop(0, nk_pad // _NBUF, outer, 0)

        for c in range(_NBUF):
            pltpu.make_async_copy(
                scr[c], out.at[pl.ds(0, _TPB)], osem[c]).wait()

    return sc_kernel


def kernel(node_feat, edge_lengths, radial_cutoff_fn, edge_index, prefactor, invr0):
    N = node_feat.shape[0]
    E = edge_index.shape[1]
    nblk = E // _BLK
    tab = jnp.pad(node_feat.reshape(N, _D), ((0, 0), (0, _DP - _D)))
    tab = tab.reshape(N // 8, 8, 3, 128).transpose(0, 2, 1, 3)
    tab = tab.reshape(N // 8 * 24, 128)
    src = edge_index[0]
    r0 = (src // 8) * 24 + (src % 8)
    idx4 = r0.reshape(nblk, _BLK // 8, 1, 8) + (jnp.arange(3, dtype=jnp.int32)
                                                * 8).reshape(1, 1, 3, 1)
    gidx = idx4.reshape(nblk * _TPB)
    inv_flat = _expand_params(invr0)
    pre_flat = _expand_params(prefactor)
    out = _make_sc_call(E, N)(tab, gidx, edge_lengths, radial_cutoff_fn,
                              inv_flat, pre_flat)
    out = out.reshape(E // 8, 3, 8, 128).transpose(0, 2, 1, 3)
    out = out.reshape(E, _DP)[:, :_D]
    return out.reshape(E, _R, _A, _C)


# speedup vs baseline: 1.0690x; 1.0690x over previous
import functools

import jax
import jax.numpy as jnp
from jax import lax
from jax.experimental import pallas as pl
from jax.experimental.pallas import tpu as pltpu
from jax.experimental.pallas import tpu_sc as plsc

_GROUPS = ((0, 1), (1, 4), (4, 10))
_R, _A, _C = 4, 10, 8
_D = _R * _A * _C
_L = 16
_NW = 32
_BLK = 128
_NBUF = 3


def _expand_params(p):
    parts = [jnp.broadcast_to(p[g][:, None, :], (_R, e - s, _C))
             for g, (s, e) in enumerate(_GROUPS)]
    return jnp.concatenate(parts, axis=1).reshape(_D)


def _make_sc_call(E, N):
    nblk = E // _BLK
    nk_max = -(-nblk // _NW)
    nk_pad = -(-nk_max // _NBUF) * _NBUF
    mesh = plsc.VectorSubcoreMesh(core_axis_name="c", subcore_axis_name="s")
    njc = _D // _L

    @functools.partial(
        pl.kernel,
        mesh=mesh,
        compiler_params=pltpu.CompilerParams(use_tc_tiling_on_sc=False),
        out_type=jax.ShapeDtypeStruct((E, _D), jnp.float32),
        scratch_types=(
            [pltpu.VMEM((1, _BLK), jnp.int32) for _ in range(_NBUF)]
            + [pltpu.VMEM((2, _BLK), jnp.float32) for _ in range(_NBUF)]
            + [pltpu.VMEM((_BLK, _D), jnp.float32) for _ in range(_NBUF)]
            + [pltpu.VMEM((_D,), jnp.float32), pltpu.VMEM((_D,), jnp.float32)]
            + [pltpu.SemaphoreType.DMA for _ in range(3 * _NBUF)]
        ),
    )
    def sc_kernel(table, src, elcf, inv, pre, out, *refs):
        pidx = refs[0:_NBUF]
        pec = refs[_NBUF:2 * _NBUF]
        rows = refs[2 * _NBUF:3 * _NBUF]
        inv_v, pre_v = refs[3 * _NBUF], refs[3 * _NBUF + 1]
        gsem = refs[3 * _NBUF + 2:3 * _NBUF + 2 + _NBUF]
        osem = refs[3 * _NBUF + 2 + _NBUF:3 * _NBUF + 2 + 2 * _NBUF]
        psem = refs[3 * _NBUF + 2 + 2 * _NBUF:]

        wid = lax.axis_index("s") * 2 + lax.axis_index("c")

        pltpu.sync_copy(inv, inv_v)
        pltpu.sync_copy(pre, pre_v)
        inv_vecs = [inv_v[pl.ds(j * _L, _L)] for j in range(njc)]
        pre_vecs = [pre_v[pl.ds(j * _L, _L)] for j in range(njc)]

        def bid_of(k):
            return wid + k * _NW

        def stage_scalars(k, b):
            base = bid_of(k) * _BLK
            pltpu.async_copy(elcf.at[:, pl.ds(base, _BLK)], pec[b], psem[b])
            pltpu.async_copy(src.at[:, pl.ds(base, _BLK)], pidx[b], psem[b])

        def wait_scalars(b):
            pltpu.make_async_copy(elcf.at[:, pl.ds(0, _BLK)], pec[b],
                                  psem[b]).wait()
            pltpu.make_async_copy(src.at[:, pl.ds(0, _BLK)], pidx[b],
                                  psem[b]).wait()

        def start_gather(b):
            pltpu.async_copy(table.at[pidx[b].at[0]], rows[b], gsem[b])

        def compute_block(k, b):
            base = bid_of(k) * _BLK

            def group_body(g, c2):
                gbase = g * _L
                el_vec = pec[b][0, pl.ds(gbase, _L)]
                cf_vec = pec[b][1, pl.ds(gbase, _L)]
                for e_l in range(_L):
                    ei = jnp.full((_L,), e_l, jnp.int32)
                    nel = -el_vec.at[ei].get(mode="promise_in_bounds")
                    cf_b = cf_vec.at[ei].get(mode="promise_in_bounds")
                    e = gbase + e_l
                    for j in range(njc):
                        sl = pl.ds(j * _L, _L)
                        s = jnp.exp(nel * inv_vecs[j]) * (pre_vecs[j] * cf_b)
                        rows[b][e, sl] = rows[b][e, sl] * s
                return c2

            lax.fori_loop(0, _BLK // _L, group_body, 0)
            pltpu.async_copy(rows[b], out.at[pl.ds(base, _BLK)], osem[b])

        stage_scalars(0, 0)
        wait_scalars(0)
        start_gather(0)

        @pl.when(bid_of(1) < nblk)
        def _():
            stage_scalars(1, 1)

        def outer(k3, carry):
            for joff in range(_NBUF):
                k = k3 * _NBUF + joff
                b = joff
                bn = (joff + 1) % _NBUF
                bn2 = (joff + 2) % _NBUF

                @pl.when(bid_of(k + 1) < nblk)
                def _(k=k, b=b, bn=bn, bn2=bn2):
                    wait_scalars(bn)

                    @pl.when(k >= 2)
                    def _():
                        pltpu.make_async_copy(
                            rows[bn], out.at[pl.ds(0, _BLK)],
                            osem[bn]).wait()

                    pltpu.async_copy(table.at[pidx[bn].at[0]], rows[bn],
                                     gsem[bn])

                    @pl.when(bid_of(k + 2) < nblk)
                    def _(k=k, bn2=bn2):
                        stage_scalars(k + 2, bn2)

                @pl.when(bid_of(k) < nblk)
                def _(k=k, b=b):
                    pltpu.make_async_copy(
                        table.at[pidx[b].at[0]], rows[b], gsem[b]).wait()
                    compute_block(k, b)
            return carry

        lax.fori_loop(0, nk_pad // _NBUF, outer, 0)

        for c in range(_NBUF):
            @pl.when(bid_of(c) < nblk)
            def _(c=c):
                pltpu.make_async_copy(
                    rows[c], out.at[pl.ds(0, _BLK)], osem[c]).wait()

    return sc_kernel


def kernel(node_feat, edge_lengths, radial_cutoff_fn, edge_index, prefactor, invr0):
    N = node_feat.shape[0]
    E = edge_index.shape[1]
    table = node_feat.reshape(N, _D)
    src = edge_index[0:1]
    elcf = jnp.stack([edge_lengths, radial_cutoff_fn])
    inv_flat = _expand_params(invr0)
    pre_flat = _expand_params(prefactor)
    out = _make_sc_call(E, N)(table, src, elcf, inv_flat, pre_flat)
    return out.reshape(E, _R, _A, _C)
